# baseline (speedup 1.0000x reference)
import functools

import jax
import jax.numpy as jnp
from jax import lax
from jax.experimental import pallas as pl
from jax.experimental.pallas import tpu as pltpu
from jax.experimental.pallas import tpu_sc as plsc


def _sc_relpos(seq: int, d: int, num_rel: int):
    max_len = (num_rel + 1) // 2
    win_start = max_len - seq
    win_rows = 2 * seq
    info = plsc.get_sparse_core_info()
    nc, ns = info.num_cores, info.num_subcores
    nw = nc * ns
    assert seq % nw == 0
    rows_per_w = seq // nw

    mesh = plsc.VectorSubcoreMesh(core_axis_name="c", subcore_axis_name="s")

    @functools.partial(
        pl.kernel,
        mesh=mesh,
        out_type=jax.ShapeDtypeStruct((seq, seq, d), jnp.float32),
        scratch_types=[
            pltpu.VMEM_SHARED((win_rows, d), jnp.float32),
            pltpu.SemaphoreType.DMA,
        ],
        compiler_params=pltpu.CompilerParams(use_tc_tiling_on_sc=False),
    )
    def body(table_hbm, out_hbm, window, sem):
        cid = lax.axis_index("c")
        sid = lax.axis_index("s")

        @pl.when(sid == 0)
        def _load():
            pltpu.sync_copy(table_hbm.at[pl.ds(win_start, win_rows)], window)

        plsc.subcore_barrier()

        wid = sid * nc + cid
        base = wid * rows_per_w
        copies = []
        for k in range(rows_per_w):
            i = base + k
            c = pltpu.make_async_copy(
                window.at[pl.ds(seq - 1 - i, seq)],
                out_hbm.at[i],
                sem,
            )
            c.start()
            copies.append(c)
        for c in copies:
            c.wait()

    return body


def kernel(x, relative_embeddings):
    seq = x.shape[0]
    d = relative_embeddings.shape[1]
    num_rel = relative_embeddings.shape[0]
    return _sc_relpos(seq, d, num_rel)(relative_embeddings)


# speedup vs baseline: 2.7600x; 2.7600x over previous
import functools

import jax
import jax.numpy as jnp
from jax import lax
from jax.experimental import pallas as pl
from jax.experimental.pallas import tpu as pltpu
from jax.experimental.pallas import tpu_sc as plsc

_NSHIFT = 8
_GROWS = 128


def _sc_relpos(seq: int, d: int, num_rel: int):
    max_len = (num_rel + 1) // 2
    win_start = max_len - seq
    win_rows = 2 * seq - _NSHIFT
    info = plsc.get_sparse_core_info()
    nc, ns, nl = info.num_cores, info.num_subcores, info.num_lanes
    assert nc == 2 and _NSHIFT % nc == 0 and ns % (_NSHIFT // nc) == 0
    spc = _NSHIFT // nc
    q_per_tile = seq // (_NSHIFT * ns)
    assert seq == _NSHIFT * ns * q_per_tile and d % nl == 0
    n_sub = -(-win_rows // _GROWS)
    tiles_per_s = ns // spc
    subs_per_tile = -(-n_sub // tiles_per_s)
    tail = win_rows - (n_sub - 1) * _GROWS

    mesh = plsc.VectorSubcoreMesh(core_axis_name="c", subcore_axis_name="s")

    @functools.partial(
        pl.kernel,
        mesh=mesh,
        out_type=jax.ShapeDtypeStruct((seq, seq, d), jnp.float32),
        scratch_types=[
            pltpu.VMEM_SHARED((spc, win_rows, d), jnp.float32),
            pltpu.VMEM((_GROWS,), jnp.int32),
            pltpu.VMEM((_GROWS, d), jnp.float32),
            pltpu.SemaphoreType.DMA,
            pltpu.SemaphoreType.DMA,
        ],
    )
    def body(table_hbm, out_hbm, wins, idx_v, rows_v, gsem, osem):
        cid = lax.axis_index("c")
        sid = lax.axis_index("s")

        s_local = sid % spc
        s_global = spc * cid + s_local
        for jj in range(subs_per_tile):
            g = (sid // spc) * subs_per_tile + jj
            row0 = win_start + s_global + g * _GROWS
            for gg in range(_GROWS // nl):
                idx_v[pl.ds(gg * nl, nl)] = row0 + gg * nl + lax.iota(jnp.int32, nl)
            pltpu.async_copy(table_hbm.at[idx_v], rows_v, gsem).wait()

            @pl.when(g < n_sub - 1)
            def _full():
                pltpu.sync_copy(
                    rows_v,
                    wins.at[s_local, pl.ds(pl.multiple_of(g * _GROWS, _GROWS), _GROWS), :],
                )

            @pl.when(g == n_sub - 1)
            def _tail():
                pltpu.sync_copy(
                    rows_v.at[pl.ds(0, tail)],
                    wins.at[s_local, pl.ds((n_sub - 1) * _GROWS, tail), :],
                )

        plsc.subcore_barrier()

        copies = []
        for qq in range(q_per_tile):
            base = _NSHIFT * (q_per_tile * sid + qq)
            off = pl.multiple_of(seq - _NSHIFT - base, _NSHIFT)
            for rr in range(spc):
                i = base + spc * (nc - 1 - cid) + rr
                sl = spc - 1 - rr
                c = pltpu.make_async_copy(
                    wins.at[sl, pl.ds(off, seq), :],
                    out_hbm.at[i],
                    osem,
                )
                c.start()
                copies.append(c)
        for c in copies:
            c.wait()

    return body


def kernel(x, relative_embeddings):
    seq = x.shape[0]
    d = relative_embeddings.shape[1]
    num_rel = relative_embeddings.shape[0]
    return _sc_relpos(seq, d, num_rel)(relative_embeddings)
